# Initial kernel scaffold; baseline (speedup 1.0000x reference)
import jax
import jax.numpy as jnp
from jax.experimental import pallas as pl


def kernel(x, edge_index, W1, b1, W2, b2, W3, b3):
    raise NotImplementedError("write your pallas kernel here")


# speedup vs baseline: 8.6487x; 8.6487x over previous
import functools

import jax
import jax.numpy as jnp
from jax import lax
from jax.experimental import pallas as pl
from jax.experimental.pallas import tpu as pltpu
from jax.experimental.pallas import tpu_sc as plsc

N = 10000
E = 160000
D = 256
DH = 128
NC = 2
NS = 16
CHUNK = 128
NCHUNK = E // CHUNK
ZC = 400
NZC = N // ZC

_mesh = plsc.VectorSubcoreMesh(core_axis_name="c", subcore_axis_name="s",
                               num_cores=NC, num_subcores=NS)


_DEG_SCRATCH = [
    pltpu.VMEM_SHARED((N, DH), jnp.float32),
    pltpu.VMEM((CHUNK,), jnp.int32),
    pltpu.VMEM((CHUNK, DH), jnp.float32),
]


def _sc_degree_body(dst_hbm, ones_hbm, zeros8_hbm, out_hbm, acc, dstbuf, onesbuf):
    c = lax.axis_index("c")
    s = lax.axis_index("s")
    w = s * NC + c
    zcnt = 1 + (s < NZC - NS).astype(jnp.int32)

    def zbody(k, _):
        j = s + k * NS
        pltpu.sync_copy(zeros8_hbm, acc.at[pl.ds(j * ZC, ZC)])
        return _

    lax.fori_loop(0, zcnt, zbody, None)
    pltpu.sync_copy(ones_hbm, onesbuf)
    plsc.subcore_barrier()
    cnt = 39 + (w < NCHUNK - 39 * 32).astype(jnp.int32)

    def body(k, _):
        base = (w + k * (NC * NS)) * CHUNK
        pltpu.sync_copy(dst_hbm.at[pl.ds(base, CHUNK)], dstbuf)
        pltpu.sync_copy(onesbuf, acc.at[dstbuf], add=True)
        return _

    lax.fori_loop(0, cnt, body, None)
    plsc.subcore_barrier()

    def dbody(k, _):
        j = s + k * NS
        pltpu.sync_copy(acc.at[pl.ds(j * ZC, ZC)],
                        out_hbm.at[pl.ds(c * N + j * ZC, ZC)])
        return _

    lax.fori_loop(0, zcnt, dbody, None)


_sc_degree = functools.partial(
    pl.kernel,
    out_type=jax.ShapeDtypeStruct((NC * N, DH), jnp.float32),
    mesh=_mesh,
    scratch_types=_DEG_SCRATCH,
)(_sc_degree_body)


_AGG_SCRATCH = [
    pltpu.VMEM_SHARED((N, DH), jnp.float32),
    pltpu.VMEM((CHUNK,), jnp.int32),
    pltpu.VMEM((CHUNK,), jnp.int32),
    pltpu.VMEM((CHUNK,), jnp.int32),
    pltpu.VMEM((CHUNK, DH), jnp.float32),
    pltpu.SemaphoreType.DMA,
]


def _sc_aggregate_body(xs_hbm, src_hbm, dst_hbm, zeros_hbm, out_hbm,
                       acc, srcbuf, adjbuf, dstbuf, rows, sem):
    c = lax.axis_index("c")
    s = lax.axis_index("s")
    off = c * N
    zcnt = 1 + (s < NZC - NS).astype(jnp.int32)

    def zbody(k, _):
        j = s + k * NS
        pltpu.sync_copy(zeros_hbm, acc.at[pl.ds(j * ZC, ZC)])
        return _

    lax.fori_loop(0, zcnt, zbody, None)
    plsc.subcore_barrier()
    cnt = 78 + (s < NCHUNK - 78 * NS).astype(jnp.int32)

    def body(k, _):
        base = (s + k * NS) * CHUNK
        pltpu.sync_copy(src_hbm.at[pl.ds(base, CHUNK)], srcbuf)
        pltpu.sync_copy(dst_hbm.at[pl.ds(base, CHUNK)], dstbuf)
        for t in range(CHUNK // 16):
            adjbuf[pl.ds(t * 16, 16)] = srcbuf[pl.ds(t * 16, 16)] + off
        pltpu.async_copy(xs_hbm.at[adjbuf], rows, sem).wait()
        pltpu.sync_copy(rows, acc.at[dstbuf], add=True)
        return _

    lax.fori_loop(0, cnt, body, None)
    plsc.subcore_barrier()

    def dbody(k, _):
        j = s + k * NS
        pltpu.sync_copy(acc.at[pl.ds(j * ZC, ZC)],
                        out_hbm.at[pl.ds(c * N + j * ZC, ZC)])
        return _

    lax.fori_loop(0, zcnt, dbody, None)


_sc_aggregate = functools.partial(
    pl.kernel,
    out_type=jax.ShapeDtypeStruct((NC * N, DH), jnp.float32),
    mesh=_mesh,
    scratch_types=_AGG_SCRATCH,
)(_sc_aggregate_body)


_R = 1000


def _mm1_body(x_ref, w_ref, parts_ref, xs_ref, dinv_ref):
    p0 = parts_ref[0, :, 0:1]
    p1 = parts_ref[1, :, 0:1]
    dinv = lax.rsqrt(p0 + p1 + 1.0)
    dinv_ref[...] = dinv
    y = jnp.dot(x_ref[...], w_ref[...], preferred_element_type=jnp.float32)
    y = y * dinv
    xs_ref[0] = y[:, :DH]
    xs_ref[1] = y[:, DH:]


def _mm_mid_body(agg_ref, xsp_ref, dinv_ref, b_ref, w_ref, xs_ref):
    dinv = dinv_ref[...]
    h = jnp.concatenate([agg_ref[0] + xsp_ref[0], agg_ref[1] + xsp_ref[1]], axis=1)
    h = jnp.maximum(dinv * h + b_ref[...], 0.0)
    y = jnp.dot(h, w_ref[...], preferred_element_type=jnp.float32) * dinv
    xs_ref[0] = y[:, :DH]
    xs_ref[1] = y[:, DH:]


def _final_body(agg_ref, xsp_ref, dinv_ref, b_ref, out_ref):
    dinv = dinv_ref[...]
    h = jnp.concatenate([agg_ref[0] + xsp_ref[0], agg_ref[1] + xsp_ref[1]], axis=1)
    out_ref[...] = dinv * h + b_ref[...]


def _mm1(x, W1, parts):
    return pl.pallas_call(
        _mm1_body,
        grid=(N // _R,),
        in_specs=[
            pl.BlockSpec((_R, D), lambda i: (i, 0)),
            pl.BlockSpec((D, D), lambda i: (0, 0)),
            pl.BlockSpec((NC, _R, DH), lambda i: (0, i, 0)),
        ],
        out_specs=[
            pl.BlockSpec((NC, _R, DH), lambda i: (0, i, 0)),
            pl.BlockSpec((_R, 1), lambda i: (i, 0)),
        ],
        out_shape=[
            jax.ShapeDtypeStruct((NC, N, DH), jnp.float32),
            jax.ShapeDtypeStruct((N, 1), jnp.float32),
        ],
    )(x, W1, parts)


def _mm_mid(agg, xsp, dinv, b, W):
    return pl.pallas_call(
        _mm_mid_body,
        grid=(N // _R,),
        in_specs=[
            pl.BlockSpec((NC, _R, DH), lambda i: (0, i, 0)),
            pl.BlockSpec((NC, _R, DH), lambda i: (0, i, 0)),
            pl.BlockSpec((_R, 1), lambda i: (i, 0)),
            pl.BlockSpec((1, D), lambda i: (0, 0)),
            pl.BlockSpec((D, D), lambda i: (0, 0)),
        ],
        out_specs=pl.BlockSpec((NC, _R, DH), lambda i: (0, i, 0)),
        out_shape=jax.ShapeDtypeStruct((NC, N, DH), jnp.float32),
    )(agg, xsp, dinv, b, W)


def _final(agg, xsp, dinv, b):
    return pl.pallas_call(
        _final_body,
        grid=(N // _R,),
        in_specs=[
            pl.BlockSpec((NC, _R, DH), lambda i: (0, i, 0)),
            pl.BlockSpec((NC, _R, DH), lambda i: (0, i, 0)),
            pl.BlockSpec((_R, 1), lambda i: (i, 0)),
            pl.BlockSpec((1, D), lambda i: (0, 0)),
        ],
        out_specs=pl.BlockSpec((_R, D), lambda i: (i, 0)),
        out_shape=jax.ShapeDtypeStruct((N, D), jnp.float32),
    )(agg, xsp, dinv, b)


def kernel(x, edge_index, W1, b1, W2, b2, W3, b3):
    src = edge_index[0].astype(jnp.int32)
    dst = edge_index[1].astype(jnp.int32)
    ones128 = jnp.ones((CHUNK, DH), jnp.float32)
    zeros128 = jnp.zeros((ZC, DH), jnp.float32)

    parts = _sc_degree(dst, ones128, zeros128).reshape(NC, N, DH)
    xs1, dinv = _mm1(x, W1, parts)
    agg1 = _sc_aggregate(xs1.reshape(NC * N, DH), src, dst, zeros128)
    xs2 = _mm_mid(agg1.reshape(NC, N, DH), xs1, dinv, b1.reshape(1, D), W2)
    agg2 = _sc_aggregate(xs2.reshape(NC * N, DH), src, dst, zeros128)
    xs3 = _mm_mid(agg2.reshape(NC, N, DH), xs2, dinv, b2.reshape(1, D), W3)
    agg3 = _sc_aggregate(xs3.reshape(NC * N, DH), src, dst, zeros128)
    return _final(agg3.reshape(NC, N, DH), xs3, dinv, b3.reshape(1, D))
